# Initial kernel scaffold; baseline (speedup 1.0000x reference)
import jax
import jax.numpy as jnp
from jax.experimental import pallas as pl


def kernel(x, scale, wg, bg, w1, b1, w2, b2):
    raise NotImplementedError("write your pallas kernel here")


# speedup vs baseline: 1.9627x; 1.9627x over previous
import jax
import jax.numpy as jnp
from jax.experimental import pallas as pl
from jax.experimental.pallas import tpu as pltpu

H = 2048
E = 16
I = 1024
ALPHA = 1.702
LIMIT = 7.0
EPS = 1e-5

T = 32
C = 4
CW = 2 * I // C
IW = CW // 2


def _moe_step(x_ref, scale_ref, wg_ref, bg_ref, w1_ref, b1_ref, w2_ref, b2_ref,
              out_ref, h_ref, gates_ref, sg_ref, sl_ref):
    e = pl.program_id(0)
    c = pl.program_id(1)

    @pl.when((e == 0) & (c == 0))
    def _routing():
        xx = x_ref[...]
        h = xx * jax.lax.rsqrt(jnp.mean(xx * xx, axis=-1, keepdims=True) + EPS)
        h = h * scale_ref[...]
        h_ref[...] = h
        logits = jnp.dot(h, wg_ref[...], preferred_element_type=jnp.float32)
        logits = logits + bg_ref[...]
        iota = jax.lax.broadcasted_iota(jnp.int32, (T, E), 1)
        m1 = jnp.max(logits, axis=-1, keepdims=True)
        i1 = jnp.min(jnp.where(logits == m1, iota, E), axis=-1, keepdims=True)
        masked = jnp.where(iota == i1, -jnp.inf, logits)
        m2 = jnp.max(masked, axis=-1, keepdims=True)
        i2 = jnp.min(jnp.where(masked == m2, iota, E), axis=-1, keepdims=True)
        p1 = 1.0 / (1.0 + jnp.exp(m2 - m1))
        gates_ref[...] = jnp.where(iota == i1, p1, 0.0) + jnp.where(iota == i2, 1.0 - p1, 0.0)
        out_ref[...] = xx
        r = jax.lax.broadcasted_iota(jnp.int32, (CW, IW), 0)
        j = jax.lax.broadcasted_iota(jnp.int32, (CW, IW), 1)
        sg_ref[...] = (r == 2 * j).astype(jnp.bfloat16)
        sl_ref[...] = (r == 2 * j + 1).astype(jnp.bfloat16)

    h = h_ref[...]
    a = jnp.dot(h.astype(jnp.bfloat16), w1_ref[0].astype(jnp.bfloat16),
                preferred_element_type=jnp.float32)
    a = (a + b1_ref[0]).astype(jnp.bfloat16)
    g = jnp.dot(a, sg_ref[...], preferred_element_type=jnp.float32)
    l = jnp.dot(a, sl_ref[...], preferred_element_type=jnp.float32)
    g = jnp.minimum(g, LIMIT)
    l = jnp.clip(l, -LIMIT, LIMIT)
    u = g * (1.0 / (1.0 + jnp.exp(-ALPHA * g))) * (l + 1.0)

    iota_e = jax.lax.broadcasted_iota(jnp.int32, (T, E), 1)
    gcol = jnp.sum(jnp.where(iota_e == e, gates_ref[...], 0.0),
                   axis=-1, keepdims=True)
    partial = jnp.dot((u * gcol).astype(jnp.bfloat16), w2_ref[0].astype(jnp.bfloat16),
                      preferred_element_type=jnp.float32)
    acc = partial

    out_ref[...] += acc + jnp.where(c == 0, 1.0, 0.0) * (gcol * b2_ref[0])


def kernel(x, scale, wg, bg, w1, b1, w2, b2):
    shape = x.shape
    x2 = x.reshape(T, H)
    y = pl.pallas_call(
        _moe_step,
        grid=(E, C),
        in_specs=[
            pl.BlockSpec((T, H), lambda e, c: (0, 0)),
            pl.BlockSpec((1, H), lambda e, c: (0, 0)),
            pl.BlockSpec((H, E), lambda e, c: (0, 0)),
            pl.BlockSpec((1, E), lambda e, c: (0, 0)),
            pl.BlockSpec((1, H, CW), lambda e, c: (e, 0, c)),
            pl.BlockSpec((1, 1, CW), lambda e, c: (e, 0, c)),
            pl.BlockSpec((1, IW, H), lambda e, c: (e, c, 0)),
            pl.BlockSpec((1, 1, H), lambda e, c: (e, 0, 0)),
        ],
        out_specs=pl.BlockSpec((T, H), lambda e, c: (0, 0)),
        out_shape=jax.ShapeDtypeStruct((T, H), jnp.float32),
        scratch_shapes=[
            pltpu.VMEM((T, H), jnp.float32),
            pltpu.VMEM((T, E), jnp.float32),
            pltpu.VMEM((CW, IW), jnp.bfloat16),
            pltpu.VMEM((CW, IW), jnp.bfloat16),
        ],
    )(x2, scale.reshape(1, H), wg, bg.reshape(1, E),
      w1, b1.reshape(E, 1, 2 * I), w2, b2.reshape(E, 1, H))
    return y.reshape(shape)


# speedup vs baseline: 2.0734x; 1.0564x over previous
import jax
import jax.numpy as jnp
from jax.experimental import pallas as pl
from jax.experimental.pallas import tpu as pltpu

H = 2048
E = 16
I = 1024
ALPHA = 1.702
LIMIT = 7.0
EPS = 1e-5

T = 32
C = 2
CW = 2 * I // C
IW = CW // 2


def _moe_step(x_ref, scale_ref, wg_ref, bg_ref, w1_ref, b1_ref, w2_ref, b2_ref,
              out_ref, h_ref, gates_ref, sg_ref, sl_ref):
    e = pl.program_id(0)
    c = pl.program_id(1)

    @pl.when((e == 0) & (c == 0))
    def _routing():
        xx = x_ref[...]
        h = xx * jax.lax.rsqrt(jnp.mean(xx * xx, axis=-1, keepdims=True) + EPS)
        h = h * scale_ref[...]
        h_ref[...] = h
        logits = jnp.dot(h, wg_ref[...], preferred_element_type=jnp.float32)
        logits = logits + bg_ref[...]
        iota = jax.lax.broadcasted_iota(jnp.int32, (T, E), 1)
        m1 = jnp.max(logits, axis=-1, keepdims=True)
        i1 = jnp.min(jnp.where(logits == m1, iota, E), axis=-1, keepdims=True)
        masked = jnp.where(iota == i1, -jnp.inf, logits)
        m2 = jnp.max(masked, axis=-1, keepdims=True)
        i2 = jnp.min(jnp.where(masked == m2, iota, E), axis=-1, keepdims=True)
        p1 = 1.0 / (1.0 + jnp.exp(m2 - m1))
        gates_ref[...] = jnp.where(iota == i1, p1, 0.0) + jnp.where(iota == i2, 1.0 - p1, 0.0)
        out_ref[...] = xx
        r = jax.lax.broadcasted_iota(jnp.int32, (CW, IW), 0)
        j = jax.lax.broadcasted_iota(jnp.int32, (CW, IW), 1)
        sg_ref[...] = (r == 2 * j).astype(jnp.bfloat16)
        sl_ref[...] = (r == 2 * j + 1).astype(jnp.bfloat16)

    h = h_ref[...]
    a = jnp.dot(h.astype(jnp.bfloat16), w1_ref[0].astype(jnp.bfloat16),
                preferred_element_type=jnp.float32)
    a = (a + b1_ref[0]).astype(jnp.bfloat16)
    g = jnp.dot(a, sg_ref[...], preferred_element_type=jnp.float32)
    l = jnp.dot(a, sl_ref[...], preferred_element_type=jnp.float32)
    g = jnp.minimum(g, LIMIT)
    l = jnp.clip(l, -LIMIT, LIMIT)
    u = g * (1.0 / (1.0 + jnp.exp(-ALPHA * g))) * (l + 1.0)

    iota_e = jax.lax.broadcasted_iota(jnp.int32, (T, E), 1)
    gcol = jnp.sum(jnp.where(iota_e == e, gates_ref[...], 0.0),
                   axis=-1, keepdims=True)
    partial = jnp.dot((u * gcol).astype(jnp.bfloat16), w2_ref[0].astype(jnp.bfloat16),
                      preferred_element_type=jnp.float32)
    acc = partial

    out_ref[...] += acc + jnp.where(c == 0, 1.0, 0.0) * (gcol * b2_ref[0])


def kernel(x, scale, wg, bg, w1, b1, w2, b2):
    shape = x.shape
    x2 = x.reshape(T, H)
    y = pl.pallas_call(
        _moe_step,
        grid=(E, C),
        in_specs=[
            pl.BlockSpec((T, H), lambda e, c: (0, 0)),
            pl.BlockSpec((1, H), lambda e, c: (0, 0)),
            pl.BlockSpec((H, E), lambda e, c: (0, 0)),
            pl.BlockSpec((1, E), lambda e, c: (0, 0)),
            pl.BlockSpec((1, H, CW), lambda e, c: (e, 0, c)),
            pl.BlockSpec((1, 1, CW), lambda e, c: (e, 0, c)),
            pl.BlockSpec((1, IW, H), lambda e, c: (e, c, 0)),
            pl.BlockSpec((1, 1, H), lambda e, c: (e, 0, 0)),
        ],
        out_specs=pl.BlockSpec((T, H), lambda e, c: (0, 0)),
        out_shape=jax.ShapeDtypeStruct((T, H), jnp.float32),
        scratch_shapes=[
            pltpu.VMEM((T, H), jnp.float32),
            pltpu.VMEM((T, E), jnp.float32),
            pltpu.VMEM((CW, IW), jnp.bfloat16),
            pltpu.VMEM((CW, IW), jnp.bfloat16),
        ],
    )(x2, scale.reshape(1, H), wg, bg.reshape(1, E),
      w1, b1.reshape(E, 1, 2 * I), w2, b2.reshape(E, 1, H))
    return y.reshape(shape)
